# baseline (speedup 1.0000x reference)
import functools

import jax
import jax.numpy as jnp
from jax import lax
from jax.experimental import pallas as pl
from jax.experimental.pallas import tpu as pltpu
from jax.experimental.pallas import tpu_sc as plsc

_NC = 2
_NS = 16
_L = 16

_EPS = 1e-5


def _linear_body(x_ref, w_ref, b_ref, o_ref):
    o_ref[...] = (
        jnp.dot(x_ref[...], w_ref[...], preferred_element_type=jnp.float32)
        + b_ref[...]
    )


def _ln_relu_body(a_ref, g_ref, bb_ref, o_ref):
    a = a_ref[...]
    mu = jnp.mean(a, axis=-1, keepdims=True)
    d = a - mu
    var = jnp.mean(d * d, axis=-1, keepdims=True)
    y = d * lax.rsqrt(var + _EPS) * g_ref[...] + bb_ref[...]
    o_ref[...] = jnp.maximum(y, 0.0)


@functools.cache
def _sc_agg_fn(B, C, F, K, BI, outer):
    rows_per = C // _NS
    mesh = plsc.VectorSubcoreMesh(core_axis_name="c", subcore_axis_name="s")

    half = BI // 2

    def body(x_hbm, h_hbm, col_hbm, row_hbm, ew_hbm, out_hbm,
             acc, col_v, row_v, ew_v, rows0, rows1, gsem0, gsem1, ssem0, ssem1):
        c = lax.axis_index("c")
        s = lax.axis_index("s")
        r0 = s * rows_per
        pltpu.sync_copy(x_hbm.at[c, pl.ds(r0, rows_per)], acc.at[pl.ds(r0, rows_per)])
        plsc.subcore_barrier()

        def gstart(i, buf, sem):
            pltpu.async_copy(h_hbm.at[col_v.at[i]], buf, sem)

        def gwait(i, buf, sem):
            pltpu.make_async_copy(h_hbm.at[col_v.at[i]], buf, sem).wait()

        def sstart(i, buf, sem):
            pltpu.async_copy(buf, acc.at[row_v.at[i]], sem, add=True)

        def swait(i, buf, sem):
            pltpu.make_async_copy(buf, acc.at[row_v.at[i]], sem).wait()

        def scale(buf, i):
            @plsc.parallel_loop(0, K, 1, unroll=8)
            def _(r):
                iv = jnp.full((_L,), i, dtype=jnp.int32)
                rv = jnp.full((_L,), r, dtype=jnp.int32)
                wv = plsc.load_gather(ew_v, [iv, rv])
                for j in range(F // _L):
                    sl = pl.ds(j * _L, _L)
                    buf[r, sl] = buf[r, sl] * wv

        def block(blk, carry):
            pltpu.sync_copy(col_hbm.at[c, s, pl.ds(blk * BI, BI)], col_v)
            pltpu.sync_copy(row_hbm.at[s, pl.ds(blk * BI, BI)], row_v)
            pltpu.sync_copy(ew_hbm.at[s, pl.ds(blk * BI, BI)], ew_v)
            gstart(0, rows0, gsem0)

            def io_body(io, carry1):
                i0 = 2 * io
                i1 = i0 + 1
                gwait(i0, rows0, gsem0)

                @pl.when(io >= 1)
                def _():
                    swait(i1 - 2, rows1, ssem1)

                gstart(i1, rows1, gsem1)
                scale(rows0, i0)
                sstart(i0, rows0, ssem0)
                gwait(i1, rows1, gsem1)
                swait(i0, rows0, ssem0)

                @pl.when(io < half - 1)
                def _():
                    gstart(i0 + 2, rows0, gsem0)

                scale(rows1, i1)
                sstart(i1, rows1, ssem1)
                return carry1

            lax.fori_loop(0, half, io_body, 0)
            swait(BI - 1, rows1, ssem1)
            return carry

        lax.fori_loop(0, outer, block, 0)
        plsc.subcore_barrier()
        pltpu.sync_copy(acc.at[pl.ds(r0, rows_per)], out_hbm.at[c, pl.ds(r0, rows_per)])

    return pl.kernel(
        body,
        out_type=jax.ShapeDtypeStruct((B, C, F), jnp.float32),
        mesh=mesh,
        compiler_params=pltpu.CompilerParams(use_tc_tiling_on_sc=False, needs_layout_passes=False),
        scratch_types=[
            pltpu.VMEM_SHARED((C, F), jnp.float32),
            pltpu.VMEM((BI, K), jnp.int32),
            pltpu.VMEM((BI, K), jnp.int32),
            pltpu.VMEM((BI, K), jnp.float32),
            pltpu.VMEM((K, F), jnp.float32),
            pltpu.VMEM((K, F), jnp.float32),
            pltpu.SemaphoreType.DMA,
            pltpu.SemaphoreType.DMA,
            pltpu.SemaphoreType.DMA,
            pltpu.SemaphoreType.DMA,
        ],
    )


def kernel(x, ei, ew, W, b, gamma, beta):
    B, C, F = x.shape
    E = ei.shape[1]
    BC = B * C
    x_flat = x.reshape(BC, F)

    RB = 2000
    grid = BC // RB
    h = pl.pallas_call(
        _linear_body,
        grid=(grid,),
        in_specs=[
            pl.BlockSpec((RB, F), lambda i: (i, 0)),
            pl.BlockSpec((F, F), lambda i: (0, 0)),
            pl.BlockSpec((1, F), lambda i: (0, 0)),
        ],
        out_specs=pl.BlockSpec((RB, F), lambda i: (i, 0)),
        out_shape=jax.ShapeDtypeStruct((BC, F), jnp.float32),
    )(x_flat, W.T, b.reshape(1, F))

    e_per = E // _NS
    K = 80
    iters = e_per // K
    BI = 50
    outer = iters // BI
    col = ei[1].reshape(1, _NS, iters, K) + (
        jnp.arange(B, dtype=ei.dtype) * C
    ).reshape(B, 1, 1, 1)
    row = ei[0].reshape(_NS, iters, K)
    ewr = ew.reshape(_NS, iters, K)

    agg = _sc_agg_fn(B, C, F, K, BI, outer)(x, h, col, row, ewr)

    out = pl.pallas_call(
        _ln_relu_body,
        grid=(grid,),
        in_specs=[
            pl.BlockSpec((RB, F), lambda i: (i, 0)),
            pl.BlockSpec((1, F), lambda i: (0, 0)),
            pl.BlockSpec((1, F), lambda i: (0, 0)),
        ],
        out_specs=pl.BlockSpec((RB, F), lambda i: (i, 0)),
        out_shape=jax.ShapeDtypeStruct((BC, F), jnp.float32),
    )(agg.reshape(BC, F), gamma.reshape(1, F), beta.reshape(1, F))
    return out.reshape(B, C, F)


# speedup vs baseline: 1.0444x; 1.0444x over previous
import functools

import jax
import jax.numpy as jnp
from jax import lax
from jax.experimental import pallas as pl
from jax.experimental.pallas import tpu as pltpu
from jax.experimental.pallas import tpu_sc as plsc

_NC = 2
_NS = 16
_L = 16

_EPS = 1e-5


def _linear_body(x_ref, w_ref, b_ref, o_ref):
    o_ref[...] = (
        jnp.dot(x_ref[...], w_ref[...], preferred_element_type=jnp.float32)
        + b_ref[...]
    )


def _ln_relu_body(a_ref, g_ref, bb_ref, o_ref):
    a = a_ref[...]
    mu = jnp.mean(a, axis=-1, keepdims=True)
    d = a - mu
    var = jnp.mean(d * d, axis=-1, keepdims=True)
    y = d * lax.rsqrt(var + _EPS) * g_ref[...] + bb_ref[...]
    o_ref[...] = jnp.maximum(y, 0.0)


@functools.cache
def _sc_agg_fn(B, C, F, K, BI, outer):
    rows_per = C // _NS
    mesh = plsc.VectorSubcoreMesh(core_axis_name="c", subcore_axis_name="s")

    N = BI * outer

    def body(x_hbm, h_hbm, pk_hbm, out_hbm,
             acc, pk, rows0, rows1, gsem0, gsem1, ssem0, ssem1, stsem):
        c = lax.axis_index("c")
        s = lax.axis_index("s")
        r0 = s * rows_per
        pltpu.sync_copy(x_hbm.at[c, pl.ds(r0, rows_per)], acc.at[pl.ds(r0, rows_per)])
        plsc.subcore_barrier()

        def gstart(v, buf, sem):
            p = (v // BI) % 2
            pltpu.async_copy(h_hbm.at[pk.at[p, 0, v % BI]], buf, sem)

        def gwait(v, buf, sem):
            p = (v // BI) % 2
            pltpu.make_async_copy(h_hbm.at[pk.at[p, 0, v % BI]], buf, sem).wait()

        def sstart(v, buf, sem):
            p = (v // BI) % 2
            pltpu.async_copy(buf, acc.at[pk.at[p, 1, v % BI]], sem, add=True)

        def swait(v, buf, sem):
            p = (v // BI) % 2
            pltpu.make_async_copy(buf, acc.at[pk.at[p, 1, v % BI]], sem).wait()

        def stg_start(m):
            pltpu.async_copy(pk_hbm.at[c, s, m], pk.at[m % 2], stsem)

        def stg_wait(m):
            pltpu.make_async_copy(pk_hbm.at[c, s, m], pk.at[m % 2], stsem).wait()

        def scale(buf, v):
            p = (v // BI) % 2
            pv = jnp.full((_L,), p, dtype=jnp.int32)
            ov = jnp.full((_L,), v % BI, dtype=jnp.int32)
            fv = jnp.full((_L,), 2, dtype=jnp.int32)

            @plsc.parallel_loop(0, K, 1, unroll=4)
            def _(r):
                rv = jnp.full((_L,), r, dtype=jnp.int32)
                wv = plsc.bitcast(plsc.load_gather(pk, [pv, fv, ov, rv]), jnp.float32)
                for j in range(F // _L):
                    sl = pl.ds(j * _L, _L)
                    buf[r, sl] = buf[r, sl] * wv

        pltpu.sync_copy(pk_hbm.at[c, s, 0], pk.at[0])
        gstart(0, rows0, gsem0)
        gstart(1, rows1, gsem1)

        def step(v, buf, gsem, ssem):
            off = v % BI
            blk = v // BI

            @pl.when(jnp.logical_and(off == 2, blk + 1 < outer))
            def _():
                stg_start(blk + 1)

            @pl.when(jnp.logical_and(off == BI - 2, blk + 1 < outer))
            def _():
                stg_wait(blk + 1)

            gwait(v, buf, gsem)
            scale(buf, v)
            sstart(v, buf, ssem)
            swait(v, buf, ssem)

            @pl.when(v + 2 < N)
            def _():
                gstart(v + 2, buf, gsem)

        def pair(t, carry):
            step(2 * t, rows0, gsem0, ssem0)
            step(2 * t + 1, rows1, gsem1, ssem1)
            return carry

        lax.fori_loop(0, N // 2, pair, 0)
        plsc.subcore_barrier()
        pltpu.sync_copy(acc.at[pl.ds(r0, rows_per)], out_hbm.at[c, pl.ds(r0, rows_per)])

    return pl.kernel(
        body,
        out_type=jax.ShapeDtypeStruct((B, C, F), jnp.float32),
        mesh=mesh,
        compiler_params=pltpu.CompilerParams(use_tc_tiling_on_sc=False, needs_layout_passes=False),
        scratch_types=[
            pltpu.VMEM_SHARED((C, F), jnp.float32),
            pltpu.VMEM((2, 3, BI, K), jnp.int32),
            pltpu.VMEM((K, F), jnp.float32),
            pltpu.VMEM((K, F), jnp.float32),
            pltpu.SemaphoreType.DMA,
            pltpu.SemaphoreType.DMA,
            pltpu.SemaphoreType.DMA,
            pltpu.SemaphoreType.DMA,
            pltpu.SemaphoreType.DMA,
        ],
    )


def kernel(x, ei, ew, W, b, gamma, beta):
    B, C, F = x.shape
    E = ei.shape[1]
    BC = B * C
    x_flat = x.reshape(BC, F)

    RB = 2000
    grid = BC // RB
    h = pl.pallas_call(
        _linear_body,
        grid=(grid,),
        in_specs=[
            pl.BlockSpec((RB, F), lambda i: (i, 0)),
            pl.BlockSpec((F, F), lambda i: (0, 0)),
            pl.BlockSpec((1, F), lambda i: (0, 0)),
        ],
        out_specs=pl.BlockSpec((RB, F), lambda i: (i, 0)),
        out_shape=jax.ShapeDtypeStruct((BC, F), jnp.float32),
    )(x_flat, W.T, b.reshape(1, F))

    e_per = E // _NS
    K = 80
    iters = e_per // K
    BI = 25
    outer = iters // BI
    col = ei[1].reshape(1, _NS, outer, BI, K) + (
        jnp.arange(B, dtype=ei.dtype) * C
    ).reshape(B, 1, 1, 1, 1)
    row = jnp.broadcast_to(ei[0].reshape(1, _NS, outer, BI, K), col.shape)
    ewi = jnp.broadcast_to(
        lax.bitcast_convert_type(ew, jnp.int32).reshape(1, _NS, outer, BI, K),
        col.shape,
    )
    pk = jnp.stack([col, row, ewi], axis=3)

    agg = _sc_agg_fn(B, C, F, K, BI, outer)(x, h, pk)

    out = pl.pallas_call(
        _ln_relu_body,
        grid=(grid,),
        in_specs=[
            pl.BlockSpec((RB, F), lambda i: (i, 0)),
            pl.BlockSpec((1, F), lambda i: (0, 0)),
            pl.BlockSpec((1, F), lambda i: (0, 0)),
        ],
        out_specs=pl.BlockSpec((RB, F), lambda i: (i, 0)),
        out_shape=jax.ShapeDtypeStruct((BC, F), jnp.float32),
    )(agg.reshape(BC, F), gamma.reshape(1, F), beta.reshape(1, F))
    return out.reshape(B, C, F)


# speedup vs baseline: 1.2055x; 1.1543x over previous
import functools

import jax
import jax.numpy as jnp
from jax import lax
from jax.experimental import pallas as pl
from jax.experimental.pallas import tpu as pltpu
from jax.experimental.pallas import tpu_sc as plsc

_NC = 2
_NS = 16
_L = 16

_EPS = 1e-5


def _linear_body(x_ref, w_ref, b_ref, o_ref):
    o_ref[...] = (
        jnp.dot(x_ref[...], w_ref[...], preferred_element_type=jnp.float32)
        + b_ref[...]
    )


def _ln_relu_body(a_ref, g_ref, bb_ref, o_ref):
    a = a_ref[...]
    mu = jnp.mean(a, axis=-1, keepdims=True)
    d = a - mu
    var = jnp.mean(d * d, axis=-1, keepdims=True)
    y = d * lax.rsqrt(var + _EPS) * g_ref[...] + bb_ref[...]
    o_ref[...] = jnp.maximum(y, 0.0)


@functools.cache
def _sc_agg_fn(B, C, F, K, BI, outer):
    rows_per = C // _NS
    mesh = plsc.VectorSubcoreMesh(core_axis_name="c", subcore_axis_name="s")

    N = BI * outer
    Fp = F // 2

    def body(x_hbm, h_hbm, pk_hbm, out_hbm,
             acc, pk, bp0, bp1, bs0, bs1, gsem0, gsem1, ssem0, ssem1, stsem):
        c = lax.axis_index("c")
        s = lax.axis_index("s")
        r0 = s * rows_per
        pltpu.sync_copy(x_hbm.at[c, pl.ds(r0, rows_per)], acc.at[pl.ds(r0, rows_per)])
        plsc.subcore_barrier()

        def gstart(v, bp, sem):
            p = (v // BI) % 2
            pltpu.async_copy(h_hbm.at[pk.at[p, 0, v % BI]], bp, sem)

        def gwait(v, bp, sem):
            p = (v // BI) % 2
            pltpu.make_async_copy(h_hbm.at[pk.at[p, 0, v % BI]], bp, sem).wait()

        def sstart(v, bs, sem):
            p = (v // BI) % 2
            pltpu.async_copy(bs, acc.at[pk.at[p, 1, v % BI]], sem, add=True)

        def swait(v, bs, sem):
            p = (v // BI) % 2
            pltpu.make_async_copy(bs, acc.at[pk.at[p, 1, v % BI]], sem).wait()

        def stg_start(m):
            pltpu.async_copy(pk_hbm.at[c, s, m], pk.at[m % 2], stsem)

        def stg_wait(m):
            pltpu.make_async_copy(pk_hbm.at[c, s, m], pk.at[m % 2], stsem).wait()

        def scale(bp, bs, v):
            p = (v // BI) % 2
            pv = jnp.full((_L,), p, dtype=jnp.int32)
            ov = jnp.full((_L,), v % BI, dtype=jnp.int32)
            fv = jnp.full((_L,), 2, dtype=jnp.int32)

            @plsc.parallel_loop(0, K, 1, unroll=4)
            def _(r):
                rv = jnp.full((_L,), r, dtype=jnp.int32)
                wv = plsc.bitcast(plsc.load_gather(pk, [pv, fv, ov, rv]), jnp.float32)
                for g in range(F // (2 * _L)):
                    pb = plsc.bitcast(bp[r, pl.ds(g * _L, _L)], jnp.bfloat16)
                    a, b2 = plsc.unpack(
                        pb,
                        format=plsc.PackFormat.INTERLEAVED,
                        preferred_element_type=jnp.float32,
                    )
                    bs[r, pl.ds(2 * g * _L, _L)] = a * wv
                    bs[r, pl.ds((2 * g + 1) * _L, _L)] = b2 * wv

        pltpu.sync_copy(pk_hbm.at[c, s, 0], pk.at[0])
        gstart(0, bp0, gsem0)
        gstart(1, bp1, gsem1)

        def step(v, bp, bs, gsem, ssem):
            off = v % BI
            blk = v // BI

            @pl.when(jnp.logical_and(off == 2, blk + 1 < outer))
            def _():
                stg_start(blk + 1)

            @pl.when(jnp.logical_and(off == BI - 2, blk + 1 < outer))
            def _():
                stg_wait(blk + 1)

            gwait(v, bp, gsem)

            @pl.when(v >= 2)
            def _():
                swait(v - 2, bs, ssem)

            scale(bp, bs, v)

            @pl.when(v + 2 < N)
            def _():
                gstart(v + 2, bp, gsem)

            sstart(v, bs, ssem)

        def pair(t, carry):
            step(2 * t, bp0, bs0, gsem0, ssem0)
            step(2 * t + 1, bp1, bs1, gsem1, ssem1)
            return carry

        lax.fori_loop(0, N // 2, pair, 0)
        swait(N - 2, bs0, ssem0)
        swait(N - 1, bs1, ssem1)
        plsc.subcore_barrier()
        pltpu.sync_copy(acc.at[pl.ds(r0, rows_per)], out_hbm.at[c, pl.ds(r0, rows_per)])

    return pl.kernel(
        body,
        out_type=jax.ShapeDtypeStruct((B, C, F), jnp.float32),
        mesh=mesh,
        compiler_params=pltpu.CompilerParams(use_tc_tiling_on_sc=False, needs_layout_passes=False),
        scratch_types=[
            pltpu.VMEM_SHARED((C, F), jnp.float32),
            pltpu.VMEM((2, 3, BI, K), jnp.int32),
            pltpu.VMEM((K, Fp), jnp.int32),
            pltpu.VMEM((K, Fp), jnp.int32),
            pltpu.VMEM((K, F), jnp.float32),
            pltpu.VMEM((K, F), jnp.float32),
            pltpu.SemaphoreType.DMA,
            pltpu.SemaphoreType.DMA,
            pltpu.SemaphoreType.DMA,
            pltpu.SemaphoreType.DMA,
            pltpu.SemaphoreType.DMA,
        ],
    )


def kernel(x, ei, ew, W, b, gamma, beta):
    B, C, F = x.shape
    E = ei.shape[1]
    BC = B * C
    x_flat = x.reshape(BC, F)

    RB = 2000
    grid = BC // RB
    h = pl.pallas_call(
        _linear_body,
        grid=(grid,),
        in_specs=[
            pl.BlockSpec((RB, F), lambda i: (i, 0)),
            pl.BlockSpec((F, F), lambda i: (0, 0)),
            pl.BlockSpec((1, F), lambda i: (0, 0)),
        ],
        out_specs=pl.BlockSpec((RB, F), lambda i: (i, 0)),
        out_shape=jax.ShapeDtypeStruct((BC, F), jnp.float32),
    )(x_flat, W.T, b.reshape(1, F))

    h_bf = h.astype(jnp.bfloat16)
    h_pairs = h_bf.reshape(BC, F // 32, 2, _L).transpose(0, 1, 3, 2)
    h_packed = lax.bitcast_convert_type(h_pairs, jnp.int32).reshape(BC, F // 2)

    e_per = E // _NS
    K = 80
    iters = e_per // K
    BI = 25
    outer = iters // BI
    col = ei[1].reshape(1, _NS, outer, BI, K) + (
        jnp.arange(B, dtype=ei.dtype) * C
    ).reshape(B, 1, 1, 1, 1)
    row = jnp.broadcast_to(ei[0].reshape(1, _NS, outer, BI, K), col.shape)
    ewi = jnp.broadcast_to(
        lax.bitcast_convert_type(ew, jnp.int32).reshape(1, _NS, outer, BI, K),
        col.shape,
    )
    pk = jnp.stack([col, row, ewi], axis=3)

    agg = _sc_agg_fn(B, C, F, K, BI, outer)(x, h_packed, pk)

    out = pl.pallas_call(
        _ln_relu_body,
        grid=(grid,),
        in_specs=[
            pl.BlockSpec((RB, F), lambda i: (i, 0)),
            pl.BlockSpec((1, F), lambda i: (0, 0)),
            pl.BlockSpec((1, F), lambda i: (0, 0)),
        ],
        out_specs=pl.BlockSpec((RB, F), lambda i: (i, 0)),
        out_shape=jax.ShapeDtypeStruct((BC, F), jnp.float32),
    )(agg.reshape(BC, F), gamma.reshape(1, F), beta.reshape(1, F))
    return out.reshape(B, C, F)
